# Initial kernel scaffold; baseline (speedup 1.0000x reference)
import jax
import jax.numpy as jnp
from jax.experimental import pallas as pl


def kernel(user, item_i, item_j, embed_user, embed_item):
    raise NotImplementedError("write your pallas kernel here")


# speedup vs baseline: 1.6364x; 1.6364x over previous
import functools

import jax
import jax.numpy as jnp
from jax import lax
from jax.experimental import pallas as pl
from jax.experimental.pallas import tpu as pltpu
from jax.experimental.pallas import tpu_sc as plsc

B = 16384
L = 20
D = 32
NC = 2
NS = 16
NW = NC * NS
ROWS_PER_W = B // NW
CB = 128
NCHUNK = ROWS_PER_W // CB
IDXW = 128
NGATHER = CB * L // IDXW


def _bpr_body(user_hbm, item_i_hbm, item_j_hbm, eu_hbm, ei_hbm,
              out_i_hbm, out_j_hbm,
              uidx_v, urows_v, iidx_v, itrows_v, outv, sem):
    wid = lax.axis_index("s") * NC + lax.axis_index("c")

    def do_chunk(c, _):
        base = wid * ROWS_PER_W + c * CB
        pltpu.sync_copy(user_hbm.at[pl.ds(base, CB)], uidx_v)
        pltpu.async_copy(eu_hbm.at[uidx_v], urows_v, sem).wait()

        wc = wid * NCHUNK + c
        for item_hbm, out_hbm in ((item_i_hbm, out_i_hbm),
                                  (item_j_hbm, out_j_hbm)):
            pltpu.sync_copy(item_hbm.at[wc], iidx_v)
            descs = []
            for g in range(NGATHER):
                descs.append(
                    pltpu.async_copy(ei_hbm.at[iidx_v.at[g]],
                                     itrows_v.at[pl.ds(g * IDXW, IDXW)],
                                     sem))
            for d in descs:
                d.wait()

            lane = lax.iota(jnp.int32, 16)

            def quad_body(q, _):
                accs = [jnp.zeros((16,), jnp.float32) for _ in range(5)]
                for rr in range(4):
                    r = q * 4 + rr
                    u_lo = urows_v[r, pl.ds(0, 16)]
                    u_hi = urows_v[r, pl.ds(16, 16)]
                    for l in range(L):
                        row = r * L + l
                        p = (itrows_v[row, pl.ds(0, 16)] * u_lo
                             + itrows_v[row, pl.ds(16, 16)] * u_hi)
                        s = jnp.sum(p)
                        vi, lk = divmod(rr * L + l, 16)
                        accs[vi] = jnp.where(lane == lk, s, accs[vi])
                for vi in range(5):
                    outv[pl.ds(q * 80 + vi * 16, 16)] = accs[vi]
                return _

            lax.fori_loop(0, CB // 4, quad_body, 0, unroll=False)
            pltpu.sync_copy(outv, out_hbm.at[pl.ds(base * L, CB * L)])
        return _

    lax.fori_loop(0, NCHUNK, do_chunk, 0, unroll=False)


@jax.jit
def kernel(user, item_i, item_j, embed_user, embed_item):
    user = user.astype(jnp.int32)
    item_i2d = item_i.astype(jnp.int32).reshape(NW * NCHUNK, NGATHER, IDXW)
    item_j2d = item_j.astype(jnp.int32).reshape(NW * NCHUNK, NGATHER, IDXW)

    mesh = plsc.VectorSubcoreMesh(core_axis_name="c", subcore_axis_name="s",
                                  num_cores=NC, num_subcores=NS)
    f = pl.kernel(
        _bpr_body,
        out_type=(jax.ShapeDtypeStruct((B * L,), jnp.float32),
                  jax.ShapeDtypeStruct((B * L,), jnp.float32)),
        mesh=mesh,
        compiler_params=pltpu.CompilerParams(needs_layout_passes=False,
                                             use_tc_tiling_on_sc=False),
        scratch_types=[
            pltpu.VMEM((CB,), jnp.int32),
            pltpu.VMEM((CB, D), jnp.float32),
            pltpu.VMEM((NGATHER, IDXW), jnp.int32),
            pltpu.VMEM((CB * L, D), jnp.float32),
            pltpu.VMEM((CB * L,), jnp.float32),
            pltpu.SemaphoreType.DMA,
        ],
    )
    out_i, out_j = f(user, item_i2d, item_j2d, embed_user, embed_item)
    return out_i.reshape(B, L), out_j.reshape(B, L)


# speedup vs baseline: 1.6469x; 1.0064x over previous
import functools

import jax
import jax.numpy as jnp
from jax import lax
from jax.experimental import pallas as pl
from jax.experimental.pallas import tpu as pltpu
from jax.experimental.pallas import tpu_sc as plsc

B = 16384
L = 20
D = 32
NC = 2
NS = 16
NW = NC * NS
ROWS_PER_W = B // NW
CB = 128
NCHUNK = ROWS_PER_W // CB
IDXW = 128
NGATHER = CB * L // IDXW


def _bpr_body(user_hbm, item_i_hbm, item_j_hbm, eu_hbm, ei_hbm,
              out_i_hbm, out_j_hbm,
              uidx_v, urows_v, iidx_v, itrows_v, outv, sem):
    wid = lax.axis_index("s") * NC + lax.axis_index("c")

    def do_chunk(c, _):
        base = wid * ROWS_PER_W + c * CB
        pltpu.sync_copy(user_hbm.at[pl.ds(base, CB)], uidx_v)
        pltpu.async_copy(eu_hbm.at[uidx_v], urows_v, sem).wait()

        for item_hbm, out_hbm in ((item_i_hbm, out_i_hbm),
                                  (item_j_hbm, out_j_hbm)):
            pltpu.sync_copy(item_hbm.at[:, pl.ds(base, CB)], iidx_v)
            descs = []
            for g in range(NGATHER):
                descs.append(
                    pltpu.async_copy(ei_hbm.at[iidx_v.at[g]],
                                     itrows_v.at[pl.ds(g * IDXW, IDXW)],
                                     sem))
            for d in descs:
                d.wait()

            lane = lax.iota(jnp.int32, 16)

            def quad_body(q, _):
                accs = [jnp.zeros((16,), jnp.float32) for _ in range(5)]
                for rr in range(4):
                    r = q * 4 + rr
                    u_lo = urows_v[r, pl.ds(0, 16)]
                    u_hi = urows_v[r, pl.ds(16, 16)]
                    for l in range(L):
                        row = l * CB + r
                        p = (itrows_v[row, pl.ds(0, 16)] * u_lo
                             + itrows_v[row, pl.ds(16, 16)] * u_hi)
                        s = jnp.sum(p)
                        vi, lk = divmod(rr * L + l, 16)
                        accs[vi] = jnp.where(lane == lk, s, accs[vi])
                for vi in range(5):
                    outv[pl.ds(q * 80 + vi * 16, 16)] = accs[vi]
                return _

            lax.fori_loop(0, CB // 4, quad_body, 0, unroll=False)
            pltpu.sync_copy(outv, out_hbm.at[pl.ds(base * L, CB * L)])
        return _

    lax.fori_loop(0, NCHUNK, do_chunk, 0, unroll=False)


@jax.jit
def kernel(user, item_i, item_j, embed_user, embed_item):
    user = user.astype(jnp.int32)
    item_i2d = item_i.astype(jnp.int32).T
    item_j2d = item_j.astype(jnp.int32).T

    mesh = plsc.VectorSubcoreMesh(core_axis_name="c", subcore_axis_name="s",
                                  num_cores=NC, num_subcores=NS)
    f = pl.kernel(
        _bpr_body,
        out_type=(jax.ShapeDtypeStruct((B * L,), jnp.float32),
                  jax.ShapeDtypeStruct((B * L,), jnp.float32)),
        mesh=mesh,
        compiler_params=pltpu.CompilerParams(needs_layout_passes=False,
                                             use_tc_tiling_on_sc=False),
        scratch_types=[
            pltpu.VMEM((CB,), jnp.int32),
            pltpu.VMEM((CB, D), jnp.float32),
            pltpu.VMEM((NGATHER, IDXW), jnp.int32),
            pltpu.VMEM((CB * L, D), jnp.float32),
            pltpu.VMEM((CB * L,), jnp.float32),
            pltpu.SemaphoreType.DMA,
        ],
    )
    out_i, out_j = f(user, item_i2d, item_j2d, embed_user, embed_item)
    return out_i.reshape(B, L), out_j.reshape(B, L)
